# baseline (speedup 1.0000x reference)
import functools

import jax
import jax.numpy as jnp
from jax import lax
from jax.experimental import pallas as pl
from jax.experimental.pallas import tpu as pltpu
from jax.experimental.pallas import tpu_sc as plsc

NC = 2
NS = 16
L = 16
NW = NC * NS

B = 4
R = 2048
COLS = 2048
ROWS = B * R
RPT = ROWS // NW
RCHUNK = 8
NCHUNK = RPT // RCHUNK

_mesh = plsc.VectorSubcoreMesh(core_axis_name="c", subcore_axis_name="s")


@functools.partial(
    pl.kernel,
    out_type=jax.ShapeDtypeStruct((B, R, COLS), jnp.float32),
    mesh=_mesh,
    scratch_types=[
        pltpu.VMEM((COLS,), jnp.int32),
        pltpu.VMEM((RCHUNK, COLS), jnp.float32),
        pltpu.VMEM((RCHUNK, COLS), jnp.float32),
        pltpu.VMEM((RCHUNK, COLS), jnp.float32),
        pltpu.VMEM((RCHUNK, COLS), jnp.float32),
        pltpu.SemaphoreType.DMA,
        pltpu.SemaphoreType.DMA,
        pltpu.SemaphoreType.DMA,
        pltpu.SemaphoreType.DMA,
        pltpu.SemaphoreType.DMA,
    ],
    compiler_params=pltpu.CompilerParams(
        needs_layout_passes=False,
        skip_device_barrier=True,
        disable_bounds_checks=True,
        disable_semaphore_checks=True,
    ),
)
def _permute_rows(x_hbm, p_hbm, out_hbm, p_v, in0, in1, out0, out1,
                  si0, si1, so0, so1, sp):
    wid = lax.axis_index("s") * NC + lax.axis_index("c")
    row_base = wid * RPT
    bi = row_base // R
    r_base = row_base % R

    def in_copy(buf, sem, r0):
        return pltpu.make_async_copy(
            x_hbm.at[bi, pl.ds(r0, RCHUNK), :], buf, sem)

    def out_copy(buf, sem, r0):
        return pltpu.make_async_copy(
            buf, out_hbm.at[bi, pl.ds(r0, RCHUNK), :], sem)

    in_copy(in0, si0, r_base).start()
    in_copy(in1, si1, r_base + RCHUNK).start()
    pltpu.async_copy(p_hbm, p_v, sp).wait()

    def gather_chunk(inb, outb):
        @plsc.parallel_loop(0, COLS // L, step=1, unroll=8)
        def col_body(j):
            idx = p_v[pl.ds(j * L, L)]
            for r in range(RCHUNK):
                rvec = jnp.full((L,), r, jnp.int32)
                outb[r, pl.ds(j * L, L)] = plsc.load_gather(
                    inb, [rvec, idx])

    bufs = ((in0, si0, out0, so0), (in1, si1, out1, so1))

    def outer(g, carry):
        for b, (inb, sib, outb, sob) in enumerate(bufs):
            ci = 2 * g + b
            r0 = r_base + ci * RCHUNK
            in_copy(inb, sib, r0).wait()

            @pl.when(ci >= 2)
            def _wait_prev_out():
                out_copy(outb, sob, r0 - 2 * RCHUNK).wait()

            gather_chunk(inb, outb)
            out_copy(outb, sob, r0).start()

            @pl.when(ci + 2 < NCHUNK)
            def _start_next_in():
                in_copy(inb, sib, r0 + 2 * RCHUNK).start()
        return carry

    lax.fori_loop(0, NCHUNK // 2, outer, 0)

    out_copy(out0, so0, r_base + (NCHUNK - 2) * RCHUNK).wait()
    out_copy(out1, so1, r_base + (NCHUNK - 1) * RCHUNK).wait()


def kernel(x, p):
    out = _permute_rows(x, p)
    return (out, 0)


# speedup vs baseline: 1.1532x; 1.1532x over previous
import functools

import jax
import jax.numpy as jnp
from jax import lax
from jax.experimental import pallas as pl
from jax.experimental.pallas import tpu as pltpu
from jax.experimental.pallas import tpu_sc as plsc

NC = 2
NS = 16
L = 16
NW = NC * NS

B = 4
R = 2048
COLS = 2048
ROWS = B * R
RPT = ROWS // NW
RCHUNK = 8
NCHUNK = RPT // RCHUNK

_mesh = plsc.VectorSubcoreMesh(core_axis_name="c", subcore_axis_name="s")


@functools.partial(
    pl.kernel,
    out_type=jax.ShapeDtypeStruct((B, R, COLS), jnp.float32),
    mesh=_mesh,
    scratch_types=[
        pltpu.VMEM((COLS,), jnp.int32),
        pltpu.VMEM((RCHUNK, COLS), jnp.float32),
        pltpu.VMEM((RCHUNK, COLS), jnp.float32),
        pltpu.VMEM((RCHUNK, COLS), jnp.float32),
        pltpu.VMEM((RCHUNK, COLS), jnp.float32),
        pltpu.VMEM((RCHUNK, COLS), jnp.float32),
        pltpu.VMEM((RCHUNK, COLS), jnp.float32),
        pltpu.SemaphoreType.DMA,
        pltpu.SemaphoreType.DMA,
        pltpu.SemaphoreType.DMA,
        pltpu.SemaphoreType.DMA,
        pltpu.SemaphoreType.DMA,
        pltpu.SemaphoreType.DMA,
        pltpu.SemaphoreType.DMA,
    ],
    compiler_params=pltpu.CompilerParams(
        needs_layout_passes=False,
        skip_device_barrier=True,
        disable_bounds_checks=True,
        disable_semaphore_checks=True,
    ),
)
def _permute_rows(x_hbm, p_hbm, out_hbm, p_v, in0, in1, in2,
                  out0, out1, out2, si0, si1, si2, so0, so1, so2, sp):
    wid = lax.axis_index("s") * NC + lax.axis_index("c")
    row_base = wid * RPT
    bi = row_base // R
    r_base = row_base % R

    def in_copy(buf, sem, r0):
        return pltpu.make_async_copy(
            x_hbm.at[bi, pl.ds(r0, RCHUNK), :], buf, sem)

    def out_copy(buf, sem, r0):
        return pltpu.make_async_copy(
            buf, out_hbm.at[bi, pl.ds(r0, RCHUNK), :], sem)

    in_copy(in0, si0, r_base).start()
    in_copy(in1, si1, r_base + RCHUNK).start()
    in_copy(in2, si2, r_base + 2 * RCHUNK).start()
    pltpu.async_copy(p_hbm, p_v, sp).wait()

    def gather_chunk(inb, outb):
        @plsc.parallel_loop(0, COLS // L, step=1, unroll=8)
        def col_body(j):
            idx = p_v[pl.ds(j * L, L)]
            for r in range(RCHUNK):
                rvec = jnp.full((L,), r, jnp.int32)
                outb[r, pl.ds(j * L, L)] = plsc.load_gather(
                    inb, [rvec, idx])

    bufs = ((in0, si0, out0, so0), (in1, si1, out1, so1),
            (in2, si2, out2, so2))
    NBUF = len(bufs)
    NITER = (NCHUNK + NBUF - 1) // NBUF

    def outer(g, carry):
        for b, (inb, sib, outb, sob) in enumerate(bufs):
            ci = NBUF * g + b
            r0 = r_base + ci * RCHUNK

            @pl.when(ci < NCHUNK)
            def _slot():
                in_copy(inb, sib, r0).wait()

                @pl.when(ci >= NBUF)
                def _wait_prev_out():
                    out_copy(outb, sob, r0 - NBUF * RCHUNK).wait()

                gather_chunk(inb, outb)
                out_copy(outb, sob, r0).start()

                @pl.when(ci + NBUF < NCHUNK)
                def _start_next_in():
                    in_copy(inb, sib, r0 + NBUF * RCHUNK).start()
        return carry

    lax.fori_loop(0, NITER, outer, 0)

    for b, (inb, sib, outb, sob) in enumerate(bufs):
        last_ci = ((NCHUNK - 1 - b) // NBUF) * NBUF + b
        out_copy(outb, sob, r_base + last_ci * RCHUNK).wait()


def kernel(x, p):
    out = _permute_rows(x, p)
    return (out, 0)
